# baseline (speedup 1.0000x reference)
import jax
import jax.numpy as jnp
import numpy as np
from jax import lax
from jax.experimental import pallas as pl
from jax.experimental.pallas import tpu as pltpu

_B, _K, _H, _W = 64, 17, 128, 128
_V = _H * _W
_G = 8
_R = _B * _K
_T = _H // 8

_TINY = np.float32(np.finfo(np.float32).tiny)
_EPS = np.float32(1e-8)
_K1 = np.int32(42)
_K2 = np.int32(0 ^ 42 ^ 0x1BD11BDA)
_ROT_A = (13, 15, 26, 6)
_ROT_B = (17, 29, 16, 24)


def _rotl(x, d):
    return lax.shift_left(x, np.int32(d)) | lax.shift_right_logical(
        x, np.int32(32 - d)
    )


def _four_rounds(x0, x1, rots):
    for r in rots:
        x0 = x0 + x1
        x1 = _rotl(x1, r)
        x1 = x0 ^ x1
    return x0, x1


def _threefry_bits(x1):
    x0 = x1
    x1 = x0 ^ _rotl(x1, _ROT_A[0])
    for r in _ROT_A[1:]:
        x0 = x0 + x1
        x1 = _rotl(x1, r)
        x1 = x0 ^ x1
    x0 = x0 + _K1
    x1 = x1 + np.int32(_K2 + 1)
    x0, x1 = _four_rounds(x0, x1, _ROT_B)
    x0 = x0 + _K2
    x1 = x1 + np.int32(2)
    x0, x1 = _four_rounds(x0, x1, _ROT_A)
    x1 = x1 + np.int32(_K1 + 3)
    x0, x1 = _four_rounds(x0, x1, _ROT_B)
    x0 = x0 + _K1
    x1 = x1 + np.int32(_K2 + 4)
    x0, x1 = _four_rounds(x0, x1, _ROT_A)
    x0 = x0 + _K2
    x1 = x1 + np.int32(5)
    return x0 ^ x1


def _sample_body(hm_ref, idx_ref, logp_ref):
    r = pl.program_id(0)
    l_full = hm_ref[0]
    m = jnp.max(l_full)
    lse = jnp.log(jnp.sum(jnp.exp(l_full - m)))

    vbase = (
        lax.broadcasted_iota(jnp.int32, (8, _W), 0) * np.int32(_W)
        + lax.broadcasted_iota(jnp.int32, (8, _W), 1)
    )
    bases = [
        (np.int32(s * _R) + r) * np.int32(_V) + np.int32(42) for s in range(_G)
    ]
    neg_inf = jnp.full((8, _W), -np.inf, jnp.float32)
    zeros_i = jnp.zeros((8, _W), jnp.int32)
    zeros_f = jnp.zeros((8, _W), jnp.float32)

    def body(t, carry):
        vmaxs, vargs, vlogs = carry
        lsl = hm_ref[0, pl.ds(t * 8, 8), :]
        vit = vbase + t * np.int32(8 * _W)
        vmaxs_n, vargs_n, vlogs_n = [], [], []
        for s in range(_G):
            bits = _threefry_bits(bases[s] + vit)
            fbits = lax.shift_right_logical(bits, np.int32(9)) | np.int32(
                0x3F800000
            )
            f = lax.bitcast_convert_type(fbits, jnp.float32) - np.float32(1.0)
            u = jnp.maximum(_TINY, f + _TINY)
            z = -jnp.log(-jnp.log(u)) + lsl
            upd = z > vmaxs[s]
            vmaxs_n.append(jnp.where(upd, z, vmaxs[s]))
            vargs_n.append(jnp.where(upd, vit, vargs[s]))
            vlogs_n.append(jnp.where(upd, lsl, vlogs[s]))
        return tuple(vmaxs_n), tuple(vargs_n), tuple(vlogs_n)

    init = (
        tuple(neg_inf for _ in range(_G)),
        tuple(zeros_i for _ in range(_G)),
        tuple(zeros_f for _ in range(_G)),
    )
    vmaxs, vargs, vlogs = lax.fori_loop(0, _T, body, init)

    lane = lax.broadcasted_iota(jnp.int32, (1, 1, _G), 2)
    idx_out = jnp.zeros((1, 1, _G), jnp.int32)
    logp_out = jnp.zeros((1, 1, _G), jnp.float32)
    for s in range(_G):
        zm = jnp.max(vmaxs[s])
        win = jnp.min(jnp.where(vmaxs[s] == zm, vargs[s], np.int32(_V)))
        lwin = jnp.sum(jnp.where(vargs[s] == win, vlogs[s], np.float32(0.0)))
        idx_out = jnp.where(lane == s, win, idx_out)
        logp_out = jnp.where(lane == s, (lwin - m) - lse, logp_out)
    idx_ref[...] = idx_out
    logp_ref[...] = logp_out


def _loss_body(idx_ref, logp_ref, out_ref):
    idx = idx_ref[...]
    logp = logp_ref[...]
    x = (idx % np.int32(_W)).astype(jnp.float32)
    y = (idx // np.int32(_W)).astype(jnp.float32)
    cx = np.float32((_W - 1) / 2.0)
    cy = np.float32((_H - 1) / 2.0)
    d = jnp.sqrt((x - cx) * (x - cx) + (y - cy) * (y - cy))
    rewards = -(jnp.sum(d, axis=1) / np.float32(_K)) / np.float32(max(_H, _W))
    rmean = jnp.mean(rewards, axis=-1, keepdims=True)
    dev = rewards - rmean
    std = jnp.sqrt(jnp.sum(dev * dev, axis=-1, keepdims=True) / np.float32(_G - 1))
    adv = dev / jnp.maximum(std, _EPS)
    adv = jnp.clip(adv, -5.0, 5.0)
    log_pi = jnp.sum(logp, axis=1)
    loss = -jnp.mean(adv * log_pi)
    reward_mean = jnp.mean(rewards)
    rdev = rewards - reward_mean
    reward_std = jnp.sqrt(jnp.sum(rdev * rdev) / np.float32(_B * _G - 1))
    adv_abs_mean = jnp.mean(jnp.abs(adv))
    lanes = lax.broadcasted_iota(jnp.int32, (1, 128), 1)
    vec = jnp.where(lanes == 0, loss, np.float32(0.0))
    vec = jnp.where(lanes == 1, reward_mean, vec)
    vec = jnp.where(lanes == 2, reward_std, vec)
    vec = jnp.where(lanes == 3, adv_abs_mean, vec)
    out_ref[...] = vec


def _run(heatmaps, interpret=False):
    hm = heatmaps.reshape(_R, _H, _W)
    idx, logp = pl.pallas_call(
        _sample_body,
        grid=(_R,),
        in_specs=[
            pl.BlockSpec((1, _H, _W), lambda r: (r, 0, 0)),
        ],
        out_specs=[
            pl.BlockSpec((1, 1, _G), lambda r: (r, 0, 0)),
            pl.BlockSpec((1, 1, _G), lambda r: (r, 0, 0)),
        ],
        out_shape=[
            jax.ShapeDtypeStruct((_R, 1, _G), jnp.int32),
            jax.ShapeDtypeStruct((_R, 1, _G), jnp.float32),
        ],
        compiler_params=pltpu.CompilerParams(
            dimension_semantics=("parallel",)
        ),
        interpret=interpret,
    )(hm)

    idx = idx.reshape(_B, _K, _G)
    logp = logp.reshape(_B, _K, _G)
    out = pl.pallas_call(
        _loss_body,
        in_specs=[
            pl.BlockSpec(idx.shape, lambda: (0, 0, 0)),
            pl.BlockSpec(logp.shape, lambda: (0, 0, 0)),
        ],
        out_specs=pl.BlockSpec((1, 128), lambda: (0, 0)),
        out_shape=jax.ShapeDtypeStruct((1, 128), jnp.float32),
        interpret=interpret,
    )(idx, logp)
    return (out[0, 0], out[0, 1], out[0, 2], out[0, 3])


def kernel(heatmaps):
    return _run(heatmaps)


# speedup vs baseline: 2.1906x; 2.1906x over previous
import jax
import jax.numpy as jnp
import numpy as np
from jax import lax
from jax.experimental import pallas as pl
from jax.experimental.pallas import tpu as pltpu

_B, _K, _H, _W = 64, 17, 128, 128
_V = _H * _W
_G = 8
_R = _B * _K
_ROWS = 4

_TINY = np.float32(np.finfo(np.float32).tiny)
_EPS = np.float32(1e-8)
_K1 = np.int32(42)
_K2 = np.int32(0 ^ 42 ^ 0x1BD11BDA)
_ROT_A = (13, 15, 26, 6)
_ROT_B = (17, 29, 16, 24)


def _rotl(x, d):
    return lax.shift_left(x, np.int32(d)) | lax.shift_right_logical(
        x, np.int32(32 - d)
    )


def _four_rounds(x0, x1, rots):
    for r in rots:
        x0 = x0 + x1
        x1 = _rotl(x1, r)
        x1 = x0 ^ x1
    return x0, x1


def _threefry_bits(x1):
    x0 = x1
    x1 = x0 ^ _rotl(x1, _ROT_A[0])
    for r in _ROT_A[1:]:
        x0 = x0 + x1
        x1 = _rotl(x1, r)
        x1 = x0 ^ x1
    x0 = x0 + _K1
    x1 = x1 + np.int32(_K2 + 1)
    x0, x1 = _four_rounds(x0, x1, _ROT_B)
    x0 = x0 + _K2
    x1 = x1 + np.int32(2)
    x0, x1 = _four_rounds(x0, x1, _ROT_A)
    x1 = x1 + np.int32(_K1 + 3)
    x0, x1 = _four_rounds(x0, x1, _ROT_B)
    x0 = x0 + _K1
    x1 = x1 + np.int32(_K2 + 4)
    x0, x1 = _four_rounds(x0, x1, _ROT_A)
    x0 = x0 + _K2
    x1 = x1 + np.int32(5)
    return x0 ^ x1


def _gumbel_plus(x1, l):
    bits = _threefry_bits(x1)
    fbits = lax.shift_right_logical(bits, np.int32(9)) | np.int32(0x3F800000)
    f = lax.bitcast_convert_type(fbits, jnp.float32) - np.float32(1.0)
    u = jnp.maximum(_TINY, f + _TINY)
    return -jnp.log(-jnp.log(u)) + l


def _sample_body(hm_ref, idx_ref, logp_ref):
    rb = pl.program_id(0)
    vi = (
        lax.broadcasted_iota(jnp.int32, (_H, _W), 0) * np.int32(_W)
        + lax.broadcasted_iota(jnp.int32, (_H, _W), 1)
    )
    sub4 = lax.broadcasted_iota(jnp.int32, (_ROWS, 1, _G), 0)
    lane = lax.broadcasted_iota(jnp.int32, (_ROWS, 1, _G), 2)
    idx_out = jnp.zeros((_ROWS, 1, _G), jnp.int32)
    logp_out = jnp.zeros((_ROWS, 1, _G), jnp.float32)

    for rr in range(_ROWS):
        r = rb * np.int32(_ROWS) + np.int32(rr)
        l = hm_ref[rr]
        m = jnp.max(l)
        lse = jnp.log(jnp.sum(jnp.exp(l - m)))

        zs = []
        for s in range(_G):
            base = (np.int32(s * _R) + r) * np.int32(_V) + np.int32(42)
            zs.append(_gumbel_plus(base + vi, l))
        zms = [jnp.max(z) for z in zs]
        masks = [z == zm for z, zm in zip(zs, zms)]
        wins = [
            jnp.min(jnp.where(msk, vi, np.int32(_V))) for msk in masks
        ]
        lwins = [
            jnp.sum(jnp.where(vi == win, l, np.float32(0.0))) for win in wins
        ]
        for s in range(_G):
            here = (sub4 == rr) & (lane == s)
            idx_out = jnp.where(here, wins[s], idx_out)
            logp_out = jnp.where(here, (lwins[s] - m) - lse, logp_out)
    idx_ref[...] = idx_out
    logp_ref[...] = logp_out


def _loss_body(idx_ref, logp_ref, out_ref):
    idx = idx_ref[...]
    logp = logp_ref[...]
    x = (idx % np.int32(_W)).astype(jnp.float32)
    y = (idx // np.int32(_W)).astype(jnp.float32)
    cx = np.float32((_W - 1) / 2.0)
    cy = np.float32((_H - 1) / 2.0)
    d = jnp.sqrt((x - cx) * (x - cx) + (y - cy) * (y - cy))
    rewards = -(jnp.sum(d, axis=1) / np.float32(_K)) / np.float32(max(_H, _W))
    rmean = jnp.mean(rewards, axis=-1, keepdims=True)
    dev = rewards - rmean
    std = jnp.sqrt(jnp.sum(dev * dev, axis=-1, keepdims=True) / np.float32(_G - 1))
    adv = dev / jnp.maximum(std, _EPS)
    adv = jnp.clip(adv, -5.0, 5.0)
    log_pi = jnp.sum(logp, axis=1)
    loss = -jnp.mean(adv * log_pi)
    reward_mean = jnp.mean(rewards)
    rdev = rewards - reward_mean
    reward_std = jnp.sqrt(jnp.sum(rdev * rdev) / np.float32(_B * _G - 1))
    adv_abs_mean = jnp.mean(jnp.abs(adv))
    lanes = lax.broadcasted_iota(jnp.int32, (1, 128), 1)
    vec = jnp.where(lanes == 0, loss, np.float32(0.0))
    vec = jnp.where(lanes == 1, reward_mean, vec)
    vec = jnp.where(lanes == 2, reward_std, vec)
    vec = jnp.where(lanes == 3, adv_abs_mean, vec)
    out_ref[...] = vec


def _run(heatmaps, interpret=False):
    hm = heatmaps.reshape(_R, _H, _W)
    idx, logp = pl.pallas_call(
        _sample_body,
        grid=(_R // _ROWS,),
        in_specs=[
            pl.BlockSpec((_ROWS, _H, _W), lambda r: (r, 0, 0)),
        ],
        out_specs=[
            pl.BlockSpec((_ROWS, 1, _G), lambda r: (r, 0, 0)),
            pl.BlockSpec((_ROWS, 1, _G), lambda r: (r, 0, 0)),
        ],
        out_shape=[
            jax.ShapeDtypeStruct((_R, 1, _G), jnp.int32),
            jax.ShapeDtypeStruct((_R, 1, _G), jnp.float32),
        ],
        compiler_params=pltpu.CompilerParams(
            dimension_semantics=("parallel",)
        ),
        interpret=interpret,
    )(hm)

    idx = idx.reshape(_B, _K, _G)
    logp = logp.reshape(_B, _K, _G)
    out = pl.pallas_call(
        _loss_body,
        in_specs=[
            pl.BlockSpec(idx.shape, lambda: (0, 0, 0)),
            pl.BlockSpec(logp.shape, lambda: (0, 0, 0)),
        ],
        out_specs=pl.BlockSpec((1, 128), lambda: (0, 0)),
        out_shape=jax.ShapeDtypeStruct((1, 128), jnp.float32),
        interpret=interpret,
    )(idx, logp)
    return (out[0, 0], out[0, 1], out[0, 2], out[0, 3])


def kernel(heatmaps):
    return _run(heatmaps)


# speedup vs baseline: 2.2058x; 1.0069x over previous
import jax
import jax.numpy as jnp
import numpy as np
from jax import lax
from jax.experimental import pallas as pl
from jax.experimental.pallas import tpu as pltpu

_B, _K, _H, _W = 64, 17, 128, 128
_V = _H * _W
_G = 8
_R = _B * _K
_ROWS = 8

_TINY = np.float32(np.finfo(np.float32).tiny)
_EPS = np.float32(1e-8)
_K1 = np.int32(42)
_K2 = np.int32(0 ^ 42 ^ 0x1BD11BDA)
_ROT_A = (13, 15, 26, 6)
_ROT_B = (17, 29, 16, 24)


def _rotl(x, d):
    return lax.shift_left(x, np.int32(d)) | lax.shift_right_logical(
        x, np.int32(32 - d)
    )


def _four_rounds(x0, x1, rots):
    for r in rots:
        x0 = x0 + x1
        x1 = _rotl(x1, r)
        x1 = x0 ^ x1
    return x0, x1


def _threefry_bits(x1):
    x0 = x1
    x1 = x0 ^ _rotl(x1, _ROT_A[0])
    for r in _ROT_A[1:]:
        x0 = x0 + x1
        x1 = _rotl(x1, r)
        x1 = x0 ^ x1
    x0 = x0 + _K1
    x1 = x1 + np.int32(_K2 + 1)
    x0, x1 = _four_rounds(x0, x1, _ROT_B)
    x0 = x0 + _K2
    x1 = x1 + np.int32(2)
    x0, x1 = _four_rounds(x0, x1, _ROT_A)
    x1 = x1 + np.int32(_K1 + 3)
    x0, x1 = _four_rounds(x0, x1, _ROT_B)
    x0 = x0 + _K1
    x1 = x1 + np.int32(_K2 + 4)
    x0, x1 = _four_rounds(x0, x1, _ROT_A)
    x0 = x0 + _K2
    x1 = x1 + np.int32(5)
    return x0 ^ x1


def _gumbel_plus(x1, l):
    bits = _threefry_bits(x1)
    fbits = lax.shift_right_logical(bits, np.int32(9)) | np.int32(0x3F800000)
    f = lax.bitcast_convert_type(fbits, jnp.float32) - np.float32(1.0)
    u = jnp.maximum(_TINY, f + _TINY)
    return -jnp.log(-jnp.log(u)) + l


def _sample_body(hm_ref, idx_ref, logp_ref):
    rb = pl.program_id(0)
    vi = (
        lax.broadcasted_iota(jnp.int32, (_H, _W), 0) * np.int32(_W)
        + lax.broadcasted_iota(jnp.int32, (_H, _W), 1)
    )
    sub4 = lax.broadcasted_iota(jnp.int32, (_ROWS, 1, _G), 0)
    lane = lax.broadcasted_iota(jnp.int32, (_ROWS, 1, _G), 2)
    idx_out = jnp.zeros((_ROWS, 1, _G), jnp.int32)
    logp_out = jnp.zeros((_ROWS, 1, _G), jnp.float32)

    for rr in range(_ROWS):
        r = rb * np.int32(_ROWS) + np.int32(rr)
        l = hm_ref[rr]
        m = jnp.max(l)
        lse = jnp.log(jnp.sum(jnp.exp(l - m)))

        zs = []
        for s in range(_G):
            base = (np.int32(s * _R) + r) * np.int32(_V) + np.int32(42)
            zs.append(_gumbel_plus(base + vi, l))
        zms = [jnp.max(z) for z in zs]
        masks = [z == zm for z, zm in zip(zs, zms)]
        wins = [
            jnp.min(jnp.where(msk, vi, np.int32(_V))) for msk in masks
        ]
        lwins = [
            jnp.sum(jnp.where(vi == win, l, np.float32(0.0))) for win in wins
        ]
        for s in range(_G):
            here = (sub4 == rr) & (lane == s)
            idx_out = jnp.where(here, wins[s], idx_out)
            logp_out = jnp.where(here, (lwins[s] - m) - lse, logp_out)
    idx_ref[...] = idx_out
    logp_ref[...] = logp_out


def _loss_body(idx_ref, logp_ref, out_ref):
    idx = idx_ref[...]
    logp = logp_ref[...]
    x = (idx % np.int32(_W)).astype(jnp.float32)
    y = (idx // np.int32(_W)).astype(jnp.float32)
    cx = np.float32((_W - 1) / 2.0)
    cy = np.float32((_H - 1) / 2.0)
    d = jnp.sqrt((x - cx) * (x - cx) + (y - cy) * (y - cy))
    rewards = -(jnp.sum(d, axis=1) / np.float32(_K)) / np.float32(max(_H, _W))
    rmean = jnp.mean(rewards, axis=-1, keepdims=True)
    dev = rewards - rmean
    std = jnp.sqrt(jnp.sum(dev * dev, axis=-1, keepdims=True) / np.float32(_G - 1))
    adv = dev / jnp.maximum(std, _EPS)
    adv = jnp.clip(adv, -5.0, 5.0)
    log_pi = jnp.sum(logp, axis=1)
    loss = -jnp.mean(adv * log_pi)
    reward_mean = jnp.mean(rewards)
    rdev = rewards - reward_mean
    reward_std = jnp.sqrt(jnp.sum(rdev * rdev) / np.float32(_B * _G - 1))
    adv_abs_mean = jnp.mean(jnp.abs(adv))
    lanes = lax.broadcasted_iota(jnp.int32, (1, 128), 1)
    vec = jnp.where(lanes == 0, loss, np.float32(0.0))
    vec = jnp.where(lanes == 1, reward_mean, vec)
    vec = jnp.where(lanes == 2, reward_std, vec)
    vec = jnp.where(lanes == 3, adv_abs_mean, vec)
    out_ref[...] = vec


def _run(heatmaps, interpret=False):
    hm = heatmaps.reshape(_R, _H, _W)
    idx, logp = pl.pallas_call(
        _sample_body,
        grid=(_R // _ROWS,),
        in_specs=[
            pl.BlockSpec((_ROWS, _H, _W), lambda r: (r, 0, 0)),
        ],
        out_specs=[
            pl.BlockSpec((_ROWS, 1, _G), lambda r: (r, 0, 0)),
            pl.BlockSpec((_ROWS, 1, _G), lambda r: (r, 0, 0)),
        ],
        out_shape=[
            jax.ShapeDtypeStruct((_R, 1, _G), jnp.int32),
            jax.ShapeDtypeStruct((_R, 1, _G), jnp.float32),
        ],
        compiler_params=pltpu.CompilerParams(
            dimension_semantics=("parallel",)
        ),
        interpret=interpret,
    )(hm)

    idx = idx.reshape(_B, _K, _G)
    logp = logp.reshape(_B, _K, _G)
    out = pl.pallas_call(
        _loss_body,
        in_specs=[
            pl.BlockSpec(idx.shape, lambda: (0, 0, 0)),
            pl.BlockSpec(logp.shape, lambda: (0, 0, 0)),
        ],
        out_specs=pl.BlockSpec((1, 128), lambda: (0, 0)),
        out_shape=jax.ShapeDtypeStruct((1, 128), jnp.float32),
        interpret=interpret,
    )(idx, logp)
    return (out[0, 0], out[0, 1], out[0, 2], out[0, 3])


def kernel(heatmaps):
    return _run(heatmaps)
